# baseline (speedup 1.0000x reference)
import functools
from functools import partial

import jax
import jax.numpy as jnp
from jax import lax
from jax.experimental import pallas as pl
from jax.experimental.pallas import tpu as pltpu
from jax.experimental.pallas import tpu_sc as plsc

_INTERPRET = False


def _sc_gather(xbf, perm, padtot):
    t, h = xbf.shape
    info = plsc.get_sparse_core_info()
    nw = info.num_cores * info.num_subcores
    rows_pw = padtot // nw
    chunk = 64
    nchunks = rows_pw // chunk
    mesh = plsc.VectorSubcoreMesh(core_axis_name="c", subcore_axis_name="s")

    @functools.partial(
        pl.kernel, mesh=mesh,
        out_type=jax.ShapeDtypeStruct((padtot, h), xbf.dtype),
        scratch_types=[
            pltpu.VMEM((chunk,), jnp.int32),
            pltpu.VMEM((chunk, h), xbf.dtype),
            pltpu.SemaphoreType.DMA,
        ],
    )
    def k(x_hbm, perm_hbm, out_hbm, idx_v, rows_v, sem):
        wid = lax.axis_index("s") * info.num_cores + lax.axis_index("c")
        base = wid * rows_pw
        for kk in range(nchunks):
            off = base + kk * chunk
            pltpu.sync_copy(perm_hbm.at[pl.ds(off, chunk)], idx_v)
            pltpu.async_copy(x_hbm.at[idx_v], rows_v, sem).wait()
            pltpu.sync_copy(rows_v, out_hbm.at[pl.ds(off, chunk)])

    return k(xbf, perm)


def _router_body(x_ref, wg_ref, bg_ref, idx_ref, gate_ref, aux_ref):
    ne, t = idx_ref.shape
    logits = lax.dot_general(
        wg_ref[...], x_ref[...], (((1,), (1,)), ((), ())),
        preferred_element_type=jnp.float32)
    logits = logits + bg_ref[...][:, :1]
    m = jnp.max(logits, axis=0, keepdims=True)
    e = jnp.exp(logits - m)
    probs = e / jnp.sum(e, axis=0, keepdims=True)

    row = lax.broadcasted_iota(jnp.int32, (ne, t), 0)
    big = jnp.int32(ne)
    m1 = jnp.max(probs, axis=0, keepdims=True)
    i1 = jnp.min(jnp.where(probs == m1, row, big), axis=0, keepdims=True)
    masked = jnp.where(row == i1, -jnp.inf, probs)
    m2 = jnp.max(masked, axis=0, keepdims=True)
    i2 = jnp.min(jnp.where(masked == m2, row, big), axis=0, keepdims=True)
    s = m1 + m2
    g1 = m1 / s
    g2 = m2 / s

    idx_ref[...] = jnp.concatenate(
        [i1, i2] + [jnp.zeros_like(i1)] * (ne - 2), axis=0)
    gate_ref[...] = jnp.concatenate(
        [g1, g2] + [jnp.zeros_like(g1)] * (ne - 2), axis=0)

    counts = jnp.sum(probs, axis=1, keepdims=True)
    fractions = counts / jnp.sum(counts)
    means = counts / jnp.float32(t)
    aux_ref[...] = jnp.float32(ne) * jnp.sum(
        fractions * means, axis=0, keepdims=True)


def _router(x, Wg, bg):
    t, h = x.shape
    ne = Wg.shape[0]
    return pl.pallas_call(
        _router_body,
        out_shape=(
            jax.ShapeDtypeStruct((ne, t), jnp.int32),
            jax.ShapeDtypeStruct((ne, t), jnp.float32),
            jax.ShapeDtypeStruct((1, 1), jnp.float32),
        ),
        interpret=_INTERPRET,
    )(x, Wg, bg.reshape(ne, 1))


def _ffn_half_body(eov_ref, xs_ref, w1_ref, b1_ref, w2_ref, *rest, last):
    hb = lax.dot_general(xs_ref[...], w1_ref[0], (((1,), (1,)), ((), ())),
                         preferred_element_type=jnp.float32)
    hb = jnp.maximum(hb + b1_ref[0], 0.0)
    part = lax.dot_general(hb, w2_ref[0], (((1,), (1,)), ((), ())),
                           preferred_element_type=jnp.float32)
    if last:
        b2_ref, g_ref, prev_ref, ys_ref = rest
        ys_ref[...] = (part + prev_ref[...] + b2_ref[0]) * g_ref[...]
    else:
        (ys_ref,) = rest
        ys_ref[...] = part


def _grouped_ffn(xs, W1, b1, W2, b2, gsort, eov, *, bm):
    padtot, h = xs.shape
    ne, e, _ = W1.shape
    e2 = e // 2
    nv = padtot // bm
    b1r = b1.reshape(ne, 1, e)
    prev = None
    for c in range(2):
        last = c == 1
        in_specs = [
            pl.BlockSpec((bm, h), lambda v, eov: (v, 0)),
            pl.BlockSpec((1, e2, h), lambda v, eov, c=c: (eov[v], c, 0)),
            pl.BlockSpec((1, 1, e2), lambda v, eov, c=c: (eov[v], 0, c)),
            pl.BlockSpec((1, h, e2), lambda v, eov, c=c: (eov[v], 0, c)),
        ]
        args = [eov, xs, W1, b1r, W2]
        if last:
            in_specs += [
                pl.BlockSpec((1, 1, h), lambda v, eov: (eov[v], 0, 0)),
                pl.BlockSpec((bm, 1), lambda v, eov: (v, 0)),
                pl.BlockSpec((bm, h), lambda v, eov: (v, 0)),
            ]
            args += [b2.reshape(ne, 1, h), gsort.reshape(padtot, 1), prev]
        prev = pl.pallas_call(
            partial(_ffn_half_body, last=last),
            grid_spec=pltpu.PrefetchScalarGridSpec(
                num_scalar_prefetch=1,
                grid=(nv,),
                in_specs=in_specs,
                out_specs=pl.BlockSpec((bm, h), lambda v, eov: (v, 0)),
            ),
            out_shape=jax.ShapeDtypeStruct((padtot, h), jnp.float32),
            interpret=_INTERPRET,
        )(*args)
    return prev


def kernel(x, Wg, bg, W1, b1, W2, b2):
    t, h = x.shape
    ne, e, _ = W1.shape
    bm = 256 if t % 256 == 0 else 8
    be = 512 if e % 512 == 0 else e
    padtot = 2 * t + ne * bm
    nv = padtot // bm

    idx8, gate8, aux = _router(x, Wg, bg)
    i0, i1 = idx8[0], idx8[1]
    g0, g1 = gate8[0], gate8[1]

    eid = jnp.concatenate([i0, i1])
    gts = jnp.concatenate([g0, g1])
    tok = jnp.concatenate([jnp.arange(t, dtype=jnp.int32)] * 2)
    onehot = (eid[:, None] == jnp.arange(ne, dtype=jnp.int32)[None, :])
    counts = jnp.sum(onehot.astype(jnp.int32), axis=0)
    padded = ((counts + bm - 1) // bm) * bm
    pad_base = jnp.concatenate(
        [jnp.zeros((1,), jnp.int32), jnp.cumsum(padded)[:-1].astype(jnp.int32)])
    rank = jnp.cumsum(onehot.astype(jnp.int32), axis=0) - 1
    rank = jnp.take_along_axis(rank, eid[:, None], axis=1)[:, 0]
    pos = pad_base[eid] + rank
    perm = jnp.zeros((padtot,), jnp.int32).at[pos].set(tok)
    gsort = jnp.zeros((padtot,), jnp.float32).at[pos].set(gts)
    vb = jnp.arange(nv, dtype=jnp.int32) * bm
    eov = jnp.sum((vb[:, None] >= pad_base[None, 1:]).astype(jnp.int32), axis=1)

    xs = _sc_gather(x, perm, padtot)
    ys = _grouped_ffn(xs, W1, b1, W2, b2, gsort, eov, bm=bm)
    out = ys[pos[:t]] + ys[pos[t:]]
    return out, aux[0, 0]


# speedup vs baseline: 1.0047x; 1.0047x over previous
import functools
from functools import partial

import jax
import jax.numpy as jnp
from jax import lax
from jax.experimental import pallas as pl
from jax.experimental.pallas import tpu as pltpu
from jax.experimental.pallas import tpu_sc as plsc

_INTERPRET = False


def _sc_gather(xbf, perm, padtot):
    t, h = xbf.shape
    info = plsc.get_sparse_core_info()
    nw = info.num_cores * info.num_subcores
    rows_pw = padtot // nw
    chunk = 16
    nchunks = rows_pw // chunk
    mesh = plsc.VectorSubcoreMesh(core_axis_name="c", subcore_axis_name="s")

    @functools.partial(
        pl.kernel, mesh=mesh,
        out_type=jax.ShapeDtypeStruct((padtot, h), xbf.dtype),
        scratch_types=[
            pltpu.VMEM((nchunks, chunk), jnp.int32),
            pltpu.VMEM((chunk, h), xbf.dtype),
            pltpu.VMEM((chunk, h), xbf.dtype),
            pltpu.SemaphoreType.DMA,
            pltpu.SemaphoreType.DMA,
        ],
    )
    def k(x_hbm, perm_hbm, out_hbm, idx_v, rows0, rows1, sem0, sem1):
        wid = lax.axis_index("s") * info.num_cores + lax.axis_index("c")
        base = wid * rows_pw
        for kk in range(nchunks):
            pltpu.sync_copy(perm_hbm.at[pl.ds(base + kk * chunk, chunk)],
                            idx_v.at[kk])
        bufs = (rows0, rows1)
        sems = (sem0, sem1)
        cps = []
        for kk in range(nchunks):
            cps.append(pltpu.async_copy(x_hbm.at[idx_v.at[kk]],
                                        bufs[kk % 2], sems[kk % 2]))
            if kk >= 1:
                cps[kk - 1].wait()
                pltpu.sync_copy(bufs[(kk - 1) % 2],
                                out_hbm.at[pl.ds(base + (kk - 1) * chunk,
                                                 chunk)])
        cps[nchunks - 1].wait()
        pltpu.sync_copy(bufs[(nchunks - 1) % 2],
                        out_hbm.at[pl.ds(base + (nchunks - 1) * chunk, chunk)])

    return k(xbf, perm)


def _sc_combine(ys, pos, t):
    padtot, h = ys.shape
    info = plsc.get_sparse_core_info()
    nw = info.num_cores * info.num_subcores
    tok_pw = t // nw
    chunk = 16
    nchunks = tok_pw // chunk
    nveck = h // 16
    mesh = plsc.VectorSubcoreMesh(core_axis_name="c", subcore_axis_name="s")

    @functools.partial(
        pl.kernel, mesh=mesh,
        out_type=jax.ShapeDtypeStruct((t, h), jnp.float32),
        scratch_types=[
            pltpu.VMEM((chunk,), jnp.int32),
            pltpu.VMEM((chunk,), jnp.int32),
            pltpu.VMEM((chunk, h), jnp.float32),
            pltpu.VMEM((chunk, h), jnp.float32),
            pltpu.SemaphoreType.DMA,
            pltpu.SemaphoreType.DMA,
        ],
    )
    def k(ys_hbm, pos_hbm, out_hbm, i0_v, i1_v, r0, r1, sem0, sem1):
        wid = lax.axis_index("s") * info.num_cores + lax.axis_index("c")
        base = wid * tok_pw
        for kk in range(nchunks):
            off = base + kk * chunk
            pltpu.sync_copy(pos_hbm.at[pl.ds(off, chunk)], i0_v)
            pltpu.sync_copy(pos_hbm.at[pl.ds(t + off, chunk)], i1_v)
            cp0 = pltpu.async_copy(ys_hbm.at[i0_v], r0, sem0)
            cp1 = pltpu.async_copy(ys_hbm.at[i1_v], r1, sem1)
            cp0.wait()
            cp1.wait()

            def add_row(j, carry):
                for kv in range(nveck):
                    sl = pl.ds(kv * 16, 16)
                    r0[j, sl] = r0[j, sl] + r1[j, sl]
                return carry

            lax.fori_loop(0, chunk, add_row, 0)
            pltpu.sync_copy(r0, out_hbm.at[pl.ds(off, chunk)])

    return k(ys, pos)


def _router_body(x_ref, wg_ref, bg_ref, idx_ref, gate_ref, aux_ref):
    ne, t = idx_ref.shape
    logits = lax.dot_general(
        wg_ref[...], x_ref[...], (((1,), (1,)), ((), ())),
        preferred_element_type=jnp.float32)
    logits = logits + bg_ref[...][:, :1]
    m = jnp.max(logits, axis=0, keepdims=True)
    e = jnp.exp(logits - m)
    probs = e / jnp.sum(e, axis=0, keepdims=True)

    row = lax.broadcasted_iota(jnp.int32, (ne, t), 0)
    big = jnp.int32(ne)
    m1 = jnp.max(probs, axis=0, keepdims=True)
    i1 = jnp.min(jnp.where(probs == m1, row, big), axis=0, keepdims=True)
    masked = jnp.where(row == i1, -jnp.inf, probs)
    m2 = jnp.max(masked, axis=0, keepdims=True)
    i2 = jnp.min(jnp.where(masked == m2, row, big), axis=0, keepdims=True)
    s = m1 + m2
    g1 = m1 / s
    g2 = m2 / s

    idx_ref[...] = jnp.concatenate(
        [i1, i2] + [jnp.zeros_like(i1)] * (ne - 2), axis=0)
    gate_ref[...] = jnp.concatenate(
        [g1, g2] + [jnp.zeros_like(g1)] * (ne - 2), axis=0)

    counts = jnp.sum(probs, axis=1, keepdims=True)
    fractions = counts / jnp.sum(counts)
    means = counts / jnp.float32(t)
    aux_ref[...] = jnp.float32(ne) * jnp.sum(
        fractions * means, axis=0, keepdims=True)


def _router(x, Wg, bg):
    t, h = x.shape
    ne = Wg.shape[0]
    return pl.pallas_call(
        _router_body,
        out_shape=(
            jax.ShapeDtypeStruct((ne, t), jnp.int32),
            jax.ShapeDtypeStruct((ne, t), jnp.float32),
            jax.ShapeDtypeStruct((1, 1), jnp.float32),
        ),
        interpret=_INTERPRET,
    )(x, Wg, bg.reshape(ne, 1))


def _ffn_half_body(eov_ref, xs_ref, w1_ref, b1_ref, w2_ref, *rest, last):
    hb = lax.dot_general(xs_ref[...], w1_ref[0], (((1,), (1,)), ((), ())),
                         preferred_element_type=jnp.float32)
    hb = jnp.maximum(hb + b1_ref[0], 0.0)
    part = lax.dot_general(hb, w2_ref[0], (((1,), (1,)), ((), ())),
                           preferred_element_type=jnp.float32)
    if last:
        b2_ref, g_ref, prev_ref, ys_ref = rest
        ys_ref[...] = (part + prev_ref[...] + b2_ref[0]) * g_ref[...]
    else:
        (ys_ref,) = rest
        ys_ref[...] = part


def _grouped_ffn(xs, W1, b1, W2, b2, gsort, eov, *, bm):
    padtot, h = xs.shape
    ne, e, _ = W1.shape
    e2 = e // 2
    nv = padtot // bm
    b1r = b1.reshape(ne, 1, e)
    prev = None
    for c in range(2):
        last = c == 1
        in_specs = [
            pl.BlockSpec((bm, h), lambda v, eov: (v, 0)),
            pl.BlockSpec((1, e2, h), lambda v, eov, c=c: (eov[v], c, 0)),
            pl.BlockSpec((1, 1, e2), lambda v, eov, c=c: (eov[v], 0, c)),
            pl.BlockSpec((1, h, e2), lambda v, eov, c=c: (eov[v], 0, c)),
        ]
        args = [eov, xs, W1, b1r, W2]
        if last:
            in_specs += [
                pl.BlockSpec((1, 1, h), lambda v, eov: (eov[v], 0, 0)),
                pl.BlockSpec((bm, 1), lambda v, eov: (v, 0)),
                pl.BlockSpec((bm, h), lambda v, eov: (v, 0)),
            ]
            args += [b2.reshape(ne, 1, h), gsort.reshape(padtot, 1), prev]
        prev = pl.pallas_call(
            partial(_ffn_half_body, last=last),
            grid_spec=pltpu.PrefetchScalarGridSpec(
                num_scalar_prefetch=1,
                grid=(nv,),
                in_specs=in_specs,
                out_specs=pl.BlockSpec((bm, h), lambda v, eov: (v, 0)),
            ),
            out_shape=jax.ShapeDtypeStruct((padtot, h), jnp.float32),
            interpret=_INTERPRET,
        )(*args)
    return prev


def kernel(x, Wg, bg, W1, b1, W2, b2):
    t, h = x.shape
    ne, e, _ = W1.shape
    bm = 256 if t % 256 == 0 else 8
    be = 512 if e % 512 == 0 else e
    padtot = 2 * t + ne * bm
    nv = padtot // bm

    idx8, gate8, aux = _router(x, Wg, bg)
    i0, i1 = idx8[0], idx8[1]
    g0, g1 = gate8[0], gate8[1]

    eid = jnp.concatenate([i0, i1])
    gts = jnp.concatenate([g0, g1])
    tok = jnp.concatenate([jnp.arange(t, dtype=jnp.int32)] * 2)
    onehot = (eid[:, None] == jnp.arange(ne, dtype=jnp.int32)[None, :])
    counts = jnp.sum(onehot.astype(jnp.int32), axis=0)
    padded = ((counts + bm - 1) // bm) * bm
    pad_base = jnp.concatenate(
        [jnp.zeros((1,), jnp.int32), jnp.cumsum(padded)[:-1].astype(jnp.int32)])
    rank = jnp.cumsum(onehot.astype(jnp.int32), axis=0) - 1
    rank = jnp.take_along_axis(rank, eid[:, None], axis=1)[:, 0]
    pos = pad_base[eid] + rank
    perm = jnp.zeros((padtot,), jnp.int32).at[pos].set(tok)
    gsort = jnp.zeros((padtot,), jnp.float32).at[pos].set(gts)
    vb = jnp.arange(nv, dtype=jnp.int32) * bm
    eov = jnp.sum((vb[:, None] >= pad_base[None, 1:]).astype(jnp.int32), axis=1)

    xs = _sc_gather(x, perm, padtot)
    ys = _grouped_ffn(xs, W1, b1, W2, b2, gsort, eov, bm=bm)
    out = _sc_combine(ys, pos, t)
    return out, aux[0, 0]


# speedup vs baseline: 1.2427x; 1.2368x over previous
import functools
from functools import partial

import jax
import jax.numpy as jnp
from jax import lax
from jax.experimental import pallas as pl
from jax.experimental.pallas import tpu as pltpu
from jax.experimental.pallas import tpu_sc as plsc

_INTERPRET = False


def _sc_gather(xbf, perm, padtot):
    t, h = xbf.shape
    info = plsc.get_sparse_core_info()
    nw = info.num_cores * info.num_subcores
    rows_pw = padtot // nw
    chunk = 16
    nchunks = rows_pw // chunk
    mesh = plsc.VectorSubcoreMesh(core_axis_name="c", subcore_axis_name="s")

    @functools.partial(
        pl.kernel, mesh=mesh,
        out_type=jax.ShapeDtypeStruct((padtot, h), xbf.dtype),
        scratch_types=[
            pltpu.VMEM((nchunks, chunk), jnp.int32),
            pltpu.VMEM((chunk, h), xbf.dtype),
            pltpu.VMEM((chunk, h), xbf.dtype),
            pltpu.SemaphoreType.DMA,
            pltpu.SemaphoreType.DMA,
        ],
    )
    def k(x_hbm, perm_hbm, out_hbm, idx_v, rows0, rows1, sem0, sem1):
        wid = lax.axis_index("s") * info.num_cores + lax.axis_index("c")
        base = wid * rows_pw
        for kk in range(nchunks):
            pltpu.sync_copy(perm_hbm.at[pl.ds(base + kk * chunk, chunk)],
                            idx_v.at[kk])
        bufs = (rows0, rows1)
        sems = (sem0, sem1)
        cps = []
        for kk in range(nchunks):
            cps.append(pltpu.async_copy(x_hbm.at[idx_v.at[kk]],
                                        bufs[kk % 2], sems[kk % 2]))
            if kk >= 1:
                cps[kk - 1].wait()
                pltpu.sync_copy(bufs[(kk - 1) % 2],
                                out_hbm.at[pl.ds(base + (kk - 1) * chunk,
                                                 chunk)])
        cps[nchunks - 1].wait()
        pltpu.sync_copy(bufs[(nchunks - 1) % 2],
                        out_hbm.at[pl.ds(base + (nchunks - 1) * chunk, chunk)])

    return k(xbf, perm)


def _sc_combine(ys, pos, t):
    padtot, h = ys.shape
    info = plsc.get_sparse_core_info()
    nw = info.num_cores * info.num_subcores
    tok_pw = t // nw
    chunk = 16
    nchunks = tok_pw // chunk
    nveck = h // 16
    mesh = plsc.VectorSubcoreMesh(core_axis_name="c", subcore_axis_name="s")

    @functools.partial(
        pl.kernel, mesh=mesh,
        out_type=jax.ShapeDtypeStruct((t, h), jnp.float32),
        scratch_types=[
            pltpu.VMEM((chunk,), jnp.int32),
            pltpu.VMEM((chunk,), jnp.int32),
            pltpu.VMEM((chunk, h), jnp.float32),
            pltpu.VMEM((chunk, h), jnp.float32),
            pltpu.SemaphoreType.DMA,
            pltpu.SemaphoreType.DMA,
        ],
    )
    def k(ys_hbm, pos_hbm, out_hbm, i0_v, i1_v, r0, r1, sem0, sem1):
        wid = lax.axis_index("s") * info.num_cores + lax.axis_index("c")
        base = wid * tok_pw
        for kk in range(nchunks):
            off = base + kk * chunk
            pltpu.sync_copy(pos_hbm.at[pl.ds(off, chunk)], i0_v)
            pltpu.sync_copy(pos_hbm.at[pl.ds(t + off, chunk)], i1_v)
            cp0 = pltpu.async_copy(ys_hbm.at[i0_v], r0, sem0)
            cp1 = pltpu.async_copy(ys_hbm.at[i1_v], r1, sem1)
            cp0.wait()
            cp1.wait()

            def add_row(j, carry):
                for kv in range(nveck):
                    sl = pl.ds(kv * 16, 16)
                    r0[j, sl] = r0[j, sl] + r1[j, sl]
                return carry

            lax.fori_loop(0, chunk, add_row, 0)
            pltpu.sync_copy(r0, out_hbm.at[pl.ds(off, chunk)])

    return k(ys, pos)


def _router_body(x_ref, wg_ref, bg_ref, idx_ref, gate_ref, aux_ref):
    ne, t = idx_ref.shape
    logits = lax.dot_general(
        wg_ref[...], x_ref[...], (((1,), (1,)), ((), ())),
        preferred_element_type=jnp.float32)
    logits = logits + bg_ref[...][:, :1]
    m = jnp.max(logits, axis=0, keepdims=True)
    e = jnp.exp(logits - m)
    probs = e / jnp.sum(e, axis=0, keepdims=True)

    row = lax.broadcasted_iota(jnp.int32, (ne, t), 0)
    big = jnp.int32(ne)
    m1 = jnp.max(probs, axis=0, keepdims=True)
    i1 = jnp.min(jnp.where(probs == m1, row, big), axis=0, keepdims=True)
    masked = jnp.where(row == i1, -jnp.inf, probs)
    m2 = jnp.max(masked, axis=0, keepdims=True)
    i2 = jnp.min(jnp.where(masked == m2, row, big), axis=0, keepdims=True)
    s = m1 + m2
    g1 = m1 / s
    g2 = m2 / s

    idx_ref[...] = jnp.concatenate(
        [i1, i2] + [jnp.zeros_like(i1)] * (ne - 2), axis=0)
    gate_ref[...] = jnp.concatenate(
        [g1, g2] + [jnp.zeros_like(g1)] * (ne - 2), axis=0)

    counts = jnp.sum(probs, axis=1, keepdims=True)
    fractions = counts / jnp.sum(counts)
    means = counts / jnp.float32(t)
    aux_ref[...] = jnp.float32(ne) * jnp.sum(
        fractions * means, axis=0, keepdims=True)


def _router(x, Wg, bg):
    t, h = x.shape
    ne = Wg.shape[0]
    return pl.pallas_call(
        _router_body,
        out_shape=(
            jax.ShapeDtypeStruct((ne, t), jnp.int32),
            jax.ShapeDtypeStruct((ne, t), jnp.float32),
            jax.ShapeDtypeStruct((1, 1), jnp.float32),
        ),
        interpret=_INTERPRET,
    )(x, Wg, bg.reshape(ne, 1))


def _ffn_half_body(eov_ref, xs_ref, w1_ref, b1_ref, w2_ref, *rest, last):
    hb = lax.dot_general(xs_ref[...], w1_ref[0], (((1,), (1,)), ((), ())),
                         preferred_element_type=jnp.float32)
    hb = jnp.maximum(hb + b1_ref[0], 0.0)
    part = lax.dot_general(hb, w2_ref[0], (((1,), (1,)), ((), ())),
                           preferred_element_type=jnp.float32)
    if last:
        b2_ref, g_ref, prev_ref, ys_ref = rest
        ys_ref[...] = (part + prev_ref[...].astype(jnp.float32)
                       + b2_ref[0]) * g_ref[...]
    else:
        (ys_ref,) = rest
        ys_ref[...] = part.astype(ys_ref.dtype)


def _grouped_ffn(xs, W1, b1, W2, b2, gsort, eov, *, bm):
    padtot, h = xs.shape
    ne, e, _ = W1.shape
    e2 = e // 2
    nv = padtot // bm
    b1r = b1.reshape(ne, 1, e)
    prev = None
    for c in range(2):
        last = c == 1
        in_specs = [
            pl.BlockSpec((bm, h), lambda v, eov: (v, 0)),
            pl.BlockSpec((1, e2, h), lambda v, eov, c=c: (eov[v], c, 0)),
            pl.BlockSpec((1, 1, e2), lambda v, eov, c=c: (eov[v], 0, c)),
            pl.BlockSpec((1, h, e2), lambda v, eov, c=c: (eov[v], 0, c)),
        ]
        args = [eov, xs, W1, b1r, W2]
        if last:
            in_specs += [
                pl.BlockSpec((1, 1, h), lambda v, eov: (eov[v], 0, 0)),
                pl.BlockSpec((bm, 1), lambda v, eov: (v, 0)),
                pl.BlockSpec((bm, h), lambda v, eov: (v, 0)),
            ]
            args += [b2.reshape(ne, 1, h), gsort.reshape(padtot, 1), prev]
        prev = pl.pallas_call(
            partial(_ffn_half_body, last=last),
            grid_spec=pltpu.PrefetchScalarGridSpec(
                num_scalar_prefetch=1,
                grid=(nv,),
                in_specs=in_specs,
                out_specs=pl.BlockSpec((bm, h), lambda v, eov: (v, 0)),
            ),
            out_shape=jax.ShapeDtypeStruct(
                (padtot, h), jnp.float32 if last else jnp.bfloat16),
            interpret=_INTERPRET,
        )(*args)
    return prev


def kernel(x, Wg, bg, W1, b1, W2, b2):
    t, h = x.shape
    ne, e, _ = W1.shape
    bm = 256 if t % 256 == 0 else 8
    be = 512 if e % 512 == 0 else e
    padtot = 2 * t + ne * bm
    nv = padtot // bm

    idx8, gate8, aux = _router(x, Wg, bg)
    i0, i1 = idx8[0], idx8[1]
    g0, g1 = gate8[0], gate8[1]

    eid = jnp.concatenate([i0, i1])
    gts = jnp.concatenate([g0, g1])
    tok = jnp.concatenate([jnp.arange(t, dtype=jnp.int32)] * 2)
    onehot = (eid[:, None] == jnp.arange(ne, dtype=jnp.int32)[None, :])
    counts = jnp.sum(onehot.astype(jnp.int32), axis=0)
    padded = ((counts + bm - 1) // bm) * bm
    pad_base = jnp.concatenate(
        [jnp.zeros((1,), jnp.int32), jnp.cumsum(padded)[:-1].astype(jnp.int32)])
    rank = jnp.cumsum(onehot.astype(jnp.int32), axis=0) - 1
    rank = jnp.take_along_axis(rank, eid[:, None], axis=1)[:, 0]
    pos = pad_base[eid] + rank
    perm = jnp.zeros((padtot,), jnp.int32).at[pos].set(tok)
    gsort = jnp.zeros((padtot,), jnp.float32).at[pos].set(gts)
    vb = jnp.arange(nv, dtype=jnp.int32) * bm
    eov = jnp.sum((vb[:, None] >= pad_base[None, 1:]).astype(jnp.int32), axis=1)

    xs = x.astype(jnp.bfloat16)[perm]
    ys = _grouped_ffn(xs, W1, b1, W2, b2, gsort, eov, bm=bm)
    out = _sc_combine(ys, pos, t)
    return out, aux[0, 0]


# speedup vs baseline: 1.2434x; 1.0006x over previous
import functools
from functools import partial

import jax
import jax.numpy as jnp
from jax import lax
from jax.experimental import pallas as pl
from jax.experimental.pallas import tpu as pltpu
from jax.experimental.pallas import tpu_sc as plsc

_INTERPRET = False


def _sc_gather(xbf, perm, padtot):
    t, h = xbf.shape
    info = plsc.get_sparse_core_info()
    nw = info.num_cores * info.num_subcores
    rows_pw = padtot // nw
    chunk = 16
    nchunks = rows_pw // chunk
    mesh = plsc.VectorSubcoreMesh(core_axis_name="c", subcore_axis_name="s")

    @functools.partial(
        pl.kernel, mesh=mesh,
        out_type=jax.ShapeDtypeStruct((padtot, h), xbf.dtype),
        scratch_types=[
            pltpu.VMEM((nchunks, chunk), jnp.int32),
            pltpu.VMEM((chunk, h), xbf.dtype),
            pltpu.VMEM((chunk, h), xbf.dtype),
            pltpu.SemaphoreType.DMA,
            pltpu.SemaphoreType.DMA,
        ],
    )
    def k(x_hbm, perm_hbm, out_hbm, idx_v, rows0, rows1, sem0, sem1):
        wid = lax.axis_index("s") * info.num_cores + lax.axis_index("c")
        base = wid * rows_pw
        for kk in range(nchunks):
            pltpu.sync_copy(perm_hbm.at[pl.ds(base + kk * chunk, chunk)],
                            idx_v.at[kk])
        bufs = (rows0, rows1)
        sems = (sem0, sem1)
        cps = []
        for kk in range(nchunks):
            cps.append(pltpu.async_copy(x_hbm.at[idx_v.at[kk]],
                                        bufs[kk % 2], sems[kk % 2]))
            if kk >= 1:
                cps[kk - 1].wait()
                pltpu.sync_copy(bufs[(kk - 1) % 2],
                                out_hbm.at[pl.ds(base + (kk - 1) * chunk,
                                                 chunk)])
        cps[nchunks - 1].wait()
        pltpu.sync_copy(bufs[(nchunks - 1) % 2],
                        out_hbm.at[pl.ds(base + (nchunks - 1) * chunk, chunk)])

    return k(xbf, perm)


def _sc_dispatch(eid, gts, *, t, ne, bm, padtot):
    a = eid.shape[0]
    info = plsc.get_sparse_core_info()
    ns = info.num_subcores
    apw = a // ns
    nvr = apw // 16
    cpw = padtot // ns
    mesh = plsc.VectorSubcoreMesh(core_axis_name="c", subcore_axis_name="s")

    @functools.partial(
        pl.kernel, mesh=mesh,
        out_type=(
            jax.ShapeDtypeStruct((a,), jnp.int32),
            jax.ShapeDtypeStruct((padtot,), jnp.int32),
            jax.ShapeDtypeStruct((padtot,), jnp.float32),
            jax.ShapeDtypeStruct((32,), jnp.int32),
        ),
        scratch_types=[
            pltpu.VMEM((apw,), jnp.int32),
            pltpu.VMEM((apw,), jnp.float32),
            pltpu.VMEM((apw // 128, 128), jnp.int32),
            pltpu.VMEM((apw,), jnp.int32),
            pltpu.VMEM((16,), jnp.int32),
            pltpu.VMEM((ns, 16), jnp.int32),
            pltpu.VMEM((32,), jnp.int32),
            pltpu.VMEM((cpw,), jnp.int32),
            pltpu.VMEM((cpw,), jnp.float32),
            pltpu.VMEM_SHARED((ns, 16), jnp.int32),
            pltpu.VMEM_SHARED((padtot,), jnp.int32),
            pltpu.VMEM_SHARED((padtot,), jnp.float32),
        ],
    )
    def k(eid_hbm, gts_hbm, pos_hbm, perm_hbm, gsort_hbm, eov_hbm,
          eid_v, gts_v, pos_v, tok_v, hist_v, histall_v, eov_v,
          zi_v, zf_v, hist_sh, perm_sh, gsort_sh):
        core = lax.axis_index("c")
        w = lax.axis_index("s")
        lane = lax.iota(jnp.int32, 16)
        zeros16 = jnp.zeros((16,), jnp.int32)

        @pl.when(core == 0)
        def _phase1():
            base = w * apw
            pltpu.sync_copy(eid_hbm.at[pl.ds(base, apw)], eid_v)
            pltpu.sync_copy(gts_hbm.at[pl.ds(base, apw)], gts_v)
            for kk in range(cpw // 16):
                sl = pl.ds(kk * 16, 16)
                zi_v[sl] = zeros16
                zf_v[sl] = jnp.zeros((16,), jnp.float32)
            pltpu.sync_copy(zi_v, perm_sh.at[pl.ds(w * cpw, cpw)])
            pltpu.sync_copy(zf_v, gsort_sh.at[pl.ds(w * cpw, cpw)])
            h = zeros16
            for i in range(nvr):
                ids = eid_v[pl.ds(16 * i, 16)]
                for e in range(ne):
                    c = jnp.sum((ids == e).astype(jnp.int32))
                    h = jnp.where(lane == e, h + c, h)
            hist_v[...] = h
            pltpu.sync_copy(hist_v, hist_sh.at[w])

        plsc.subcore_barrier()

        @pl.when(core == 0)
        def _phase2():
            pltpu.sync_copy(hist_sh, histall_v)
            counts = zeros16
            pre = zeros16
            for w2 in range(ns):
                row = histall_v[w2, :]
                counts = counts + row
                pre = pre + jnp.where(w2 < w, row, zeros16)
            pc = ((counts + (bm - 1)) // bm) * bm
            pad_base = jnp.cumsum(pc) - pc
            mybase = pad_base + pre
            bs = [jnp.sum(jnp.where(lane == e, mybase, zeros16))
                  for e in range(ne)]
            for i in range(nvr):
                ids = eid_v[pl.ds(16 * i, 16)]
                posv = zeros16
                for e in range(ne):
                    m = ids == e
                    mi = m.astype(jnp.int32)
                    r = jnp.cumsum(mi)
                    posv = jnp.where(m, bs[e] + r - 1, posv)
                    bs[e] = bs[e] + jnp.sum(mi)
                pos_v[i // 8, pl.ds((i % 8) * 16, 16)] = posv
                gi = w * apw + 16 * i + lane
                tok_v[pl.ds(16 * i, 16)] = jnp.where(gi >= t, gi - t, gi)
            for j in range(apw // 128):
                sl = pl.ds(j * 128, 128)
                pltpu.sync_copy(tok_v.at[sl], perm_sh.at[pos_v.at[j]])
                pltpu.sync_copy(gts_v.at[sl], gsort_sh.at[pos_v.at[j]])
                pltpu.sync_copy(pos_v.at[j],
                                pos_hbm.at[pl.ds(w * apw + j * 128, 128)])

            @pl.when(w == 0)
            def _eov():
                for i2 in range(2):
                    vb = (lane + 16 * i2) * bm
                    ev = zeros16
                    for e in range(1, ne):
                        pb = jnp.sum(jnp.where(lane == e, pad_base, zeros16))
                        ev = ev + (vb >= pb).astype(jnp.int32)
                    eov_v[pl.ds(16 * i2, 16)] = ev
                pltpu.sync_copy(eov_v, eov_hbm)

        plsc.subcore_barrier()

        @pl.when(core == 0)
        def _phase3():
            sl = pl.ds(w * cpw, cpw)
            pltpu.sync_copy(perm_sh.at[sl], perm_hbm.at[sl])
            pltpu.sync_copy(gsort_sh.at[sl], gsort_hbm.at[sl])

    return k(eid, gts)


def _sc_combine(ys, pos, t):
    padtot, h = ys.shape
    info = plsc.get_sparse_core_info()
    nw = info.num_cores * info.num_subcores
    tok_pw = t // nw
    chunk = 16
    nchunks = tok_pw // chunk
    nveck = h // 16
    mesh = plsc.VectorSubcoreMesh(core_axis_name="c", subcore_axis_name="s")

    @functools.partial(
        pl.kernel, mesh=mesh,
        out_type=jax.ShapeDtypeStruct((t, h), jnp.float32),
        scratch_types=[
            pltpu.VMEM((chunk,), jnp.int32),
            pltpu.VMEM((chunk,), jnp.int32),
            pltpu.VMEM((chunk, h), jnp.float32),
            pltpu.VMEM((chunk, h), jnp.float32),
            pltpu.SemaphoreType.DMA,
            pltpu.SemaphoreType.DMA,
        ],
    )
    def k(ys_hbm, pos_hbm, out_hbm, i0_v, i1_v, r0, r1, sem0, sem1):
        wid = lax.axis_index("s") * info.num_cores + lax.axis_index("c")
        base = wid * tok_pw
        for kk in range(nchunks):
            off = base + kk * chunk
            pltpu.sync_copy(pos_hbm.at[pl.ds(off, chunk)], i0_v)
            pltpu.sync_copy(pos_hbm.at[pl.ds(t + off, chunk)], i1_v)
            cp0 = pltpu.async_copy(ys_hbm.at[i0_v], r0, sem0)
            cp1 = pltpu.async_copy(ys_hbm.at[i1_v], r1, sem1)
            cp0.wait()
            cp1.wait()

            def add_row(j, carry):
                for kv in range(nveck):
                    sl = pl.ds(kv * 16, 16)
                    r0[j, sl] = r0[j, sl] + r1[j, sl]
                return carry

            lax.fori_loop(0, chunk, add_row, 0)
            pltpu.sync_copy(r0, out_hbm.at[pl.ds(off, chunk)])

    return k(ys, pos)


def _router_body(x_ref, wg_ref, bg_ref, idx_ref, gate_ref, aux_ref):
    ne, t = idx_ref.shape
    logits = lax.dot_general(
        wg_ref[...], x_ref[...], (((1,), (1,)), ((), ())),
        preferred_element_type=jnp.float32)
    logits = logits + bg_ref[...][:, :1]
    m = jnp.max(logits, axis=0, keepdims=True)
    e = jnp.exp(logits - m)
    probs = e / jnp.sum(e, axis=0, keepdims=True)

    row = lax.broadcasted_iota(jnp.int32, (ne, t), 0)
    big = jnp.int32(ne)
    m1 = jnp.max(probs, axis=0, keepdims=True)
    i1 = jnp.min(jnp.where(probs == m1, row, big), axis=0, keepdims=True)
    masked = jnp.where(row == i1, -jnp.inf, probs)
    m2 = jnp.max(masked, axis=0, keepdims=True)
    i2 = jnp.min(jnp.where(masked == m2, row, big), axis=0, keepdims=True)
    s = m1 + m2
    g1 = m1 / s
    g2 = m2 / s

    idx_ref[...] = jnp.concatenate(
        [i1, i2] + [jnp.zeros_like(i1)] * (ne - 2), axis=0)
    gate_ref[...] = jnp.concatenate(
        [g1, g2] + [jnp.zeros_like(g1)] * (ne - 2), axis=0)

    counts = jnp.sum(probs, axis=1, keepdims=True)
    fractions = counts / jnp.sum(counts)
    means = counts / jnp.float32(t)
    aux_ref[...] = jnp.float32(ne) * jnp.sum(
        fractions * means, axis=0, keepdims=True)


def _router(x, Wg, bg):
    t, h = x.shape
    ne = Wg.shape[0]
    return pl.pallas_call(
        _router_body,
        out_shape=(
            jax.ShapeDtypeStruct((ne, t), jnp.int32),
            jax.ShapeDtypeStruct((ne, t), jnp.float32),
            jax.ShapeDtypeStruct((1, 1), jnp.float32),
        ),
        interpret=_INTERPRET,
    )(x, Wg, bg.reshape(ne, 1))


def _ffn_half_body(eov_ref, xs_ref, w1_ref, b1_ref, w2_ref, *rest, last):
    hb = lax.dot_general(xs_ref[...], w1_ref[0], (((1,), (1,)), ((), ())),
                         preferred_element_type=jnp.float32)
    hb = jnp.maximum(hb + b1_ref[0], 0.0)
    part = lax.dot_general(hb, w2_ref[0], (((1,), (1,)), ((), ())),
                           preferred_element_type=jnp.float32)
    if last:
        b2_ref, g_ref, prev_ref, ys_ref = rest
        ys_ref[...] = (part + prev_ref[...].astype(jnp.float32)
                       + b2_ref[0]) * g_ref[...]
    else:
        (ys_ref,) = rest
        ys_ref[...] = part.astype(ys_ref.dtype)


def _grouped_ffn(xs, W1, b1, W2, b2, gsort, eov, *, bm):
    padtot, h = xs.shape
    ne, e, _ = W1.shape
    e2 = e // 2
    nv = padtot // bm
    b1r = b1.reshape(ne, 1, e)
    prev = None
    for c in range(2):
        last = c == 1
        in_specs = [
            pl.BlockSpec((bm, h), lambda v, eov: (v, 0)),
            pl.BlockSpec((1, e2, h), lambda v, eov, c=c: (eov[v], c, 0)),
            pl.BlockSpec((1, 1, e2), lambda v, eov, c=c: (eov[v], 0, c)),
            pl.BlockSpec((1, h, e2), lambda v, eov, c=c: (eov[v], 0, c)),
        ]
        args = [eov, xs, W1, b1r, W2]
        if last:
            in_specs += [
                pl.BlockSpec((1, 1, h), lambda v, eov: (eov[v], 0, 0)),
                pl.BlockSpec((bm, 1), lambda v, eov: (v, 0)),
                pl.BlockSpec((bm, h), lambda v, eov: (v, 0)),
            ]
            args += [b2.reshape(ne, 1, h), gsort.reshape(padtot, 1), prev]
        prev = pl.pallas_call(
            partial(_ffn_half_body, last=last),
            grid_spec=pltpu.PrefetchScalarGridSpec(
                num_scalar_prefetch=1,
                grid=(nv,),
                in_specs=in_specs,
                out_specs=pl.BlockSpec((bm, h), lambda v, eov: (v, 0)),
            ),
            out_shape=jax.ShapeDtypeStruct(
                (padtot, h), jnp.float32 if last else jnp.bfloat16),
            interpret=_INTERPRET,
        )(*args)
    return prev


def kernel(x, Wg, bg, W1, b1, W2, b2):
    t, h = x.shape
    ne, e, _ = W1.shape
    bm = 256 if t % 256 == 0 else 8
    be = 512 if e % 512 == 0 else e
    padtot = 2 * t + ne * bm
    nv = padtot // bm

    idx8, gate8, aux = _router(x, Wg, bg)
    i0, i1 = idx8[0], idx8[1]
    g0, g1 = gate8[0], gate8[1]

    eid = jnp.concatenate([i0, i1])
    gts = jnp.concatenate([g0, g1])
    tok = jnp.concatenate([jnp.arange(t, dtype=jnp.int32)] * 2)
    onehot = (eid[:, None] == jnp.arange(ne, dtype=jnp.int32)[None, :])
    counts = jnp.sum(onehot.astype(jnp.int32), axis=0)
    padded = ((counts + bm - 1) // bm) * bm
    pad_base = jnp.concatenate(
        [jnp.zeros((1,), jnp.int32), jnp.cumsum(padded)[:-1].astype(jnp.int32)])
    rank = jnp.cumsum(onehot.astype(jnp.int32), axis=0) - 1
    rank = jnp.take_along_axis(rank, eid[:, None], axis=1)[:, 0]
    pos = pad_base[eid] + rank
    perm = jnp.zeros((padtot,), jnp.int32).at[pos].set(tok)
    gsort = jnp.zeros((padtot,), jnp.float32).at[pos].set(gts)
    vb = jnp.arange(nv, dtype=jnp.int32) * bm
    eov = jnp.sum((vb[:, None] >= pad_base[None, 1:]).astype(jnp.int32), axis=1)

    xs = x.astype(jnp.bfloat16)[perm]
    ys = _grouped_ffn(xs, W1, b1, W2, b2, gsort, eov, bm=bm)
    out = _sc_combine(ys, pos, t)
    return out, aux[0, 0]
